# baseline (speedup 1.0000x reference)
import functools

import jax
import jax.numpy as jnp
from jax import lax
from jax.experimental import pallas as pl
from jax.experimental.pallas import tpu as pltpu
from jax.experimental.pallas import tpu_sc as plsc

D = 300
DP = 304
B = 4096 * 200
NC, NS = 2, 16
NW = NC * NS
BPW = B // NW
C = 128
NCHUNK = BPW // C
NPAIR = NCHUNK // 2

_mesh = plsc.VectorSubcoreMesh(core_axis_name="c", subcore_axis_name="s")


@functools.partial(
    pl.kernel,
    out_type=jax.ShapeDtypeStruct((B, DP), jnp.float32),
    mesh=_mesh,
    compiler_params=pltpu.CompilerParams(use_tc_tiling_on_sc=False),
    scratch_types=[
        pltpu.VMEM((BPW,), jnp.int32),
        pltpu.VMEM((C, DP), jnp.float32),
        pltpu.VMEM((C, DP), jnp.float32),
        pltpu.SemaphoreType.DMA,
        pltpu.SemaphoreType.DMA,
        pltpu.SemaphoreType.DMA,
        pltpu.SemaphoreType.DMA,
    ],
)
def _gather_kernel(idx_hbm, table_hbm, out_hbm,
                   idx_v, rows0, rows1, g0, g1, o0, o1):
    wid = lax.axis_index("s") * NC + lax.axis_index("c")
    base = wid * BPW
    pltpu.sync_copy(idx_hbm.at[pl.ds(base, BPW)], idx_v)

    def gather(g, rows, sem):
        return pltpu.make_async_copy(
            table_hbm.at[idx_v.at[pl.ds(g * C, C)]], rows, sem)

    def store(g, rows, sem):
        return pltpu.make_async_copy(
            rows, out_hbm.at[pl.ds(base + g * C, C)], sem)

    gather(0, rows0, g0).start()
    gather(1, rows1, g1).start()

    def pair(i, _):
        a = 2 * i
        gather(a, rows0, g0).wait()
        store(a, rows0, o0).start()
        gather(a + 1, rows1, g1).wait()
        store(a + 1, rows1, o1).start()
        store(a, rows0, o0).wait()
        gather(a + 2, rows0, g0).start()
        store(a + 1, rows1, o1).wait()
        gather(a + 3, rows1, g1).start()
        return 0

    lax.fori_loop(0, NPAIR - 1, pair, 0)

    last = NCHUNK - 2
    gather(last, rows0, g0).wait()
    store(last, rows0, o0).start()
    gather(last + 1, rows1, g1).wait()
    store(last + 1, rows1, o1).start()
    store(last, rows0, o0).wait()
    store(last + 1, rows1, o1).wait()


def kernel(inputs, table):
    idx = inputs.reshape(-1).astype(jnp.int32)
    table_p = jnp.pad(table, ((0, 0), (0, DP - D)))
    out = _gather_kernel(idx, table_p)
    return out[:, :D].reshape(inputs.shape[0], inputs.shape[1], D)
